# baseline (speedup 1.0000x reference)
import functools

import jax
import jax.numpy as jnp
from jax import lax
from jax.experimental import pallas as pl
from jax.experimental.pallas import tpu as pltpu
from jax.experimental.pallas import tpu_sc as plsc

_V = 1000
_D = 128
_B = 4096
_T = 200
_NW = 32
_ROWS_PER_W = (_B * _T) // _NW
_CHUNK = 128
_GPW = 2
_WCHUNK = _CHUNK * _GPW
_NGATHER = _ROWS_PER_W // _CHUNK
_NCHUNK = _ROWS_PER_W // _WCHUNK
_NBUF = 2
_LAG = 1


def _emb_body(idx_hbm, table_hbm, out_hbm, tbl_sh, idx_v, rows_v, gsem, wsem):
    cid = lax.axis_index("c")
    sid = lax.axis_index("s")
    wid = sid * 2 + cid
    out_base = wid * _ROWS_PER_W

    @pl.when(sid == 0)
    def _():
        pltpu.sync_copy(table_hbm, tbl_sh)

    plsc.subcore_barrier()

    pltpu.sync_copy(idx_hbm.at[wid], idx_v)

    def gather_issue(i, j):
        for h in range(_GPW):
            pltpu.async_copy(
                table_hbm.at[idx_v.at[_GPW * i + h]],
                rows_v.at[j, pl.ds(h * _CHUNK, _CHUNK)],
                gsem,
            )

    def gather_drain(j):
        for h in range(_GPW):
            pltpu.make_async_copy(
                table_hbm.at[pl.ds(0, _CHUNK)],
                rows_v.at[j, pl.ds(h * _CHUNK, _CHUNK)],
                gsem,
            ).wait()

    def wb_issue(g, j):
        pltpu.async_copy(
            rows_v.at[j], out_hbm.at[pl.ds(out_base + g * _WCHUNK, _WCHUNK)], wsem
        )

    def wb_drain(j):
        pltpu.make_async_copy(
            rows_v.at[j], out_hbm.at[pl.ds(out_base, _WCHUNK)], wsem
        ).wait()

    for i in range(_NBUF):
        gather_issue(i, i)
        if i >= _LAG:
            g = i - _LAG
            gather_drain(g % _NBUF)
            wb_issue(g, g % _NBUF)

    def outer(o, carry):
        for j in range(_NBUF):
            i = _NBUF + o * _NBUF + j
            wb_drain(j)
            gather_issue(i, j)
            g = i - _LAG
            gather_drain((i - _LAG) % _NBUF)
            wb_issue(g, (i - _LAG) % _NBUF)
        return carry

    lax.fori_loop(0, (_NCHUNK - _NBUF) // _NBUF, outer, 0)

    for g in range(_NCHUNK - _LAG, _NCHUNK):
        gather_drain(g % _NBUF)
        wb_issue(g, g % _NBUF)
    for j in range(_NBUF):
        wb_drain(j)


_emb = functools.partial(
    pl.kernel,
    mesh=plsc.VectorSubcoreMesh(core_axis_name="c", subcore_axis_name="s"),
    out_type=jax.ShapeDtypeStruct((_B * _T, _D), jnp.float32),
    scratch_types=[
        pltpu.MemorySpace.VMEM_SHARED((_V, _D), jnp.float32),
        pltpu.VMEM((_NGATHER, _CHUNK), jnp.int32),
        pltpu.VMEM((_NBUF, _WCHUNK, _D), jnp.float32),
        pltpu.SemaphoreType.DMA,
        pltpu.SemaphoreType.DMA,
    ],
)(_emb_body)


def kernel(vocab_ids, table):
    idx = vocab_ids.reshape(_NW, _NGATHER, _CHUNK).astype(jnp.int32)
    out = _emb(idx, table)
    return out.reshape(_B, _T, _D)


# speedup vs baseline: 1.8560x; 1.8560x over previous
import functools

import jax
import jax.numpy as jnp
from jax import lax
from jax.experimental import pallas as pl
from jax.experimental.pallas import tpu as pltpu
from jax.experimental.pallas import tpu_sc as plsc

_V = 1000
_D = 128
_B = 4096
_T = 200
_NW = 32
_ROWS_PER_W = (_B * _T) // _NW
_CHUNK = 128
_NCHUNK = _ROWS_PER_W // _CHUNK
_NBUF = 4
_LAG = 2
_HBM_SLOT = 3


def _emb_body(idx_hbm, table_hbm, out_hbm, tbl_sh, idx_v, rows_v, gsem, hsem, wsem):
    cid = lax.axis_index("c")
    sid = lax.axis_index("s")
    wid = sid * 2 + cid
    out_base = wid * _ROWS_PER_W

    @pl.when(sid == 0)
    def _():
        pltpu.sync_copy(table_hbm, tbl_sh)

    plsc.subcore_barrier()

    pltpu.sync_copy(idx_hbm.at[wid], idx_v)

    def gather_issue(i, j):
        if j == _HBM_SLOT:
            pltpu.async_copy(table_hbm.at[idx_v.at[i]], rows_v.at[j], hsem)
        else:
            pltpu.async_copy(tbl_sh.at[idx_v.at[i]], rows_v.at[j], gsem)

    def gather_drain(j):
        if j == _HBM_SLOT:
            pltpu.make_async_copy(
                table_hbm.at[pl.ds(0, _CHUNK)], rows_v.at[j], hsem
            ).wait()
        else:
            pltpu.make_async_copy(
                tbl_sh.at[pl.ds(0, _CHUNK)], rows_v.at[j], gsem
            ).wait()

    def wb_issue(g, j):
        pltpu.async_copy(
            rows_v.at[j], out_hbm.at[pl.ds(out_base + g * _CHUNK, _CHUNK)], wsem
        )

    def wb_drain(j):
        pltpu.make_async_copy(
            rows_v.at[j], out_hbm.at[pl.ds(out_base, _CHUNK)], wsem
        ).wait()

    for i in range(_NBUF):
        gather_issue(i, i)
        if i >= _LAG:
            g = i - _LAG
            gather_drain(g % _NBUF)
            wb_issue(g, g % _NBUF)

    def outer(o, carry):
        for j in range(_NBUF):
            i = _NBUF + o * _NBUF + j
            wb_drain(j)
            gather_issue(i, j)
            jg = (j - _LAG) % _NBUF
            gather_drain(jg)
            wb_issue(i - _LAG, jg)
        return carry

    lax.fori_loop(0, (_NCHUNK - _NBUF) // _NBUF, outer, 0)

    for g in range(_NCHUNK - _LAG, _NCHUNK):
        gather_drain(g % _NBUF)
        wb_issue(g, g % _NBUF)
    for j in range(_NBUF):
        wb_drain(j)


_emb = functools.partial(
    pl.kernel,
    mesh=plsc.VectorSubcoreMesh(core_axis_name="c", subcore_axis_name="s"),
    out_type=jax.ShapeDtypeStruct((_B * _T, _D), jnp.float32),
    scratch_types=[
        pltpu.MemorySpace.VMEM_SHARED((_V, _D), jnp.float32),
        pltpu.VMEM((_NCHUNK, _CHUNK), jnp.int32),
        pltpu.VMEM((_NBUF, _CHUNK, _D), jnp.float32),
        pltpu.SemaphoreType.DMA,
        pltpu.SemaphoreType.DMA,
        pltpu.SemaphoreType.DMA,
    ],
)(_emb_body)


def kernel(vocab_ids, table):
    idx = vocab_ids.reshape(_NW, _NCHUNK, _CHUNK).astype(jnp.int32)
    out = _emb(idx, table)
    return out.reshape(_B, _T, _D)


# speedup vs baseline: 1.8716x; 1.0084x over previous
import functools

import jax
import jax.numpy as jnp
from jax import lax
from jax.experimental import pallas as pl
from jax.experimental.pallas import tpu as pltpu
from jax.experimental.pallas import tpu_sc as plsc

_V = 1000
_D = 128
_B = 4096
_T = 200
_NW = 32
_ROWS_PER_W = (_B * _T) // _NW
_CHUNK = 128
_NCHUNK = _ROWS_PER_W // _CHUNK
_NGROUP = _NCHUNK // 4


def _emb_body(idx_hbm, table_hbm, out_hbm, tbl_sh, idx_v, rows_a, rows_h,
              gsem, hsem, wasem, whsem):
    cid = lax.axis_index("c")
    sid = lax.axis_index("s")
    wid = sid * 2 + cid
    out_base = wid * _ROWS_PER_W

    @pl.when(sid == 0)
    def _():
        pltpu.sync_copy(table_hbm, tbl_sh)

    plsc.subcore_barrier()

    pltpu.sync_copy(idx_hbm.at[wid], idx_v)

    def ga_issue(i, k):
        pltpu.async_copy(tbl_sh.at[idx_v.at[i]], rows_a.at[k], gsem)

    def ga_drain(k):
        pltpu.make_async_copy(tbl_sh.at[pl.ds(0, _CHUNK)], rows_a.at[k], gsem).wait()

    def wa_issue(i, k):
        pltpu.async_copy(
            rows_a.at[k], out_hbm.at[pl.ds(out_base + i * _CHUNK, _CHUNK)], wasem
        )

    def wa_drain(k):
        pltpu.make_async_copy(
            rows_a.at[k], out_hbm.at[pl.ds(out_base, _CHUNK)], wasem
        ).wait()

    def gh_issue(i, s):
        pltpu.async_copy(table_hbm.at[idx_v.at[i]], rows_h.at[s], hsem)

    def gh_drain(s):
        pltpu.make_async_copy(
            table_hbm.at[pl.ds(0, _CHUNK)], rows_h.at[s], hsem
        ).wait()

    def wh_issue(i, s):
        pltpu.async_copy(
            rows_h.at[s], out_hbm.at[pl.ds(out_base + i * _CHUNK, _CHUNK)], whsem
        )

    def wh_drain(s):
        pltpu.make_async_copy(
            rows_h.at[s], out_hbm.at[pl.ds(out_base, _CHUNK)], whsem
        ).wait()

    def run_group(o, so, drain_wh, drain_wa, issue_next_h, first_a=False):
        so1 = 1 - so
        if drain_wh:
            wh_drain(so1)
        if issue_next_h:
            gh_issue(4 * (o + 1) + 3, so1)
        gh_drain(so)
        wh_issue(4 * o + 3, so)
        for k in range(3):
            i = 4 * o + k
            if drain_wa:
                wa_drain(k)
            ga_issue(i, k)
            km1 = (k - 1) % 3
            if not (first_a and k == 0):
                if first_a:
                    ip = k - 1
                else:
                    ip = i - 1 if k >= 1 else i - 2
                ga_drain(km1)
                wa_issue(ip, km1)

    gh_issue(3, 0)
    run_group(0, 0, drain_wh=False, drain_wa=False, issue_next_h=True,
              first_a=True)
    run_group(1, 1, drain_wh=True, drain_wa=True, issue_next_h=True)

    def outer(p, carry):
        for q in range(2):
            o = 2 + 2 * p + q
            so = q
            so1 = 1 - so
            wh_drain(so1)
            gh_issue(4 * (o + 1) + 3, so1)
            gh_drain(so)
            wh_issue(4 * o + 3, so)
            for k in range(3):
                i = 4 * o + k
                wa_drain(k)
                ga_issue(i, k)
                km1 = (k - 1) % 3
                ip = i - 1 if k >= 1 else i - 2
                ga_drain(km1)
                wa_issue(ip, km1)
        return carry

    lax.fori_loop(0, 23, outer, 0)

    run_group(48, 0, drain_wh=True, drain_wa=True, issue_next_h=True)
    run_group(49, 1, drain_wh=True, drain_wa=True, issue_next_h=False)

    ga_drain(2)
    wa_issue(198, 2)

    for k in range(3):
        wa_drain(k)
    wh_drain(1)


_emb = functools.partial(
    pl.kernel,
    mesh=plsc.VectorSubcoreMesh(core_axis_name="c", subcore_axis_name="s"),
    out_type=jax.ShapeDtypeStruct((_B * _T, _D), jnp.float32),
    scratch_types=[
        pltpu.MemorySpace.VMEM_SHARED((_V, _D), jnp.float32),
        pltpu.VMEM((_NCHUNK, _CHUNK), jnp.int32),
        pltpu.VMEM((3, _CHUNK, _D), jnp.float32),
        pltpu.VMEM((2, _CHUNK, _D), jnp.float32),
        pltpu.SemaphoreType.DMA,
        pltpu.SemaphoreType.DMA,
        pltpu.SemaphoreType.DMA,
        pltpu.SemaphoreType.DMA,
    ],
)(_emb_body)


def kernel(vocab_ids, table):
    idx = vocab_ids.reshape(_NW, _NCHUNK, _CHUNK).astype(jnp.int32)
    out = _emb(idx, table)
    return out.reshape(_B, _T, _D)


# speedup vs baseline: 2.4735x; 1.3216x over previous
import functools

import jax
import jax.numpy as jnp
from jax import lax
from jax.experimental import pallas as pl
from jax.experimental.pallas import tpu as pltpu
from jax.experimental.pallas import tpu_sc as plsc

_V = 1000
_D = 128
_B = 4096
_T = 200
_NW = 32
_ROWS_PER_W = (_B * _T) // _NW
_CHUNK = 128
_NCHUNK = _ROWS_PER_W // _CHUNK
_NBUF = 4
_LAG = 2
_HBM_SLOT = -1


def _emb_body(idx_hbm, table_hbm, out_hbm, tbl_sh, idx_v, rows_v, gsem, hsem, wsem):
    cid = lax.axis_index("c")
    sid = lax.axis_index("s")
    wid = sid * 2 + cid
    out_base = wid * _ROWS_PER_W

    @pl.when(sid == 0)
    def _():
        pltpu.sync_copy(table_hbm, tbl_sh)

    plsc.subcore_barrier()

    pltpu.sync_copy(idx_hbm.at[wid], idx_v)

    def gather_issue(i, j):
        if j == _HBM_SLOT:
            pltpu.async_copy(table_hbm.at[idx_v.at[i]], rows_v.at[j], hsem)
        else:
            pltpu.async_copy(tbl_sh.at[idx_v.at[i]], rows_v.at[j], gsem)

    def gather_drain(j):
        if j == _HBM_SLOT:
            pltpu.make_async_copy(
                table_hbm.at[pl.ds(0, _CHUNK)], rows_v.at[j], hsem
            ).wait()
        else:
            pltpu.make_async_copy(
                tbl_sh.at[pl.ds(0, _CHUNK)], rows_v.at[j], gsem
            ).wait()

    def wb_issue(g, j):
        pltpu.async_copy(
            rows_v.at[j], out_hbm.at[pl.ds(out_base + g * _CHUNK, _CHUNK)], wsem
        )

    def wb_drain(j):
        pltpu.make_async_copy(
            rows_v.at[j], out_hbm.at[pl.ds(out_base, _CHUNK)], wsem
        ).wait()

    for i in range(_NBUF):
        gather_issue(i, i)
        if i >= _LAG:
            g = i - _LAG
            gather_drain(g % _NBUF)
            wb_issue(g, g % _NBUF)

    def outer(o, carry):
        for j in range(_NBUF):
            i = _NBUF + o * _NBUF + j
            wb_drain(j)
            gather_issue(i, j)
            jg = (j - _LAG) % _NBUF
            gather_drain(jg)
            wb_issue(i - _LAG, jg)
        return carry

    lax.fori_loop(0, (_NCHUNK - _NBUF) // _NBUF, outer, 0)

    for g in range(_NCHUNK - _LAG, _NCHUNK):
        gather_drain(g % _NBUF)
        wb_issue(g, g % _NBUF)
    for j in range(_NBUF):
        wb_drain(j)


_emb = functools.partial(
    pl.kernel,
    mesh=plsc.VectorSubcoreMesh(core_axis_name="c", subcore_axis_name="s"),
    out_type=jax.ShapeDtypeStruct((_B * _T, _D), jnp.float32),
    scratch_types=[
        pltpu.MemorySpace.VMEM_SHARED((_V, _D), jnp.float32),
        pltpu.VMEM((_NCHUNK, _CHUNK), jnp.int32),
        pltpu.VMEM((_NBUF, _CHUNK, _D), jnp.float32),
        pltpu.SemaphoreType.DMA,
        pltpu.SemaphoreType.DMA,
        pltpu.SemaphoreType.DMA,
    ],
)(_emb_body)


def kernel(vocab_ids, table):
    idx = vocab_ids.reshape(_NW, _NCHUNK, _CHUNK).astype(jnp.int32)
    out = _emb(idx, table)
    return out.reshape(_B, _T, _D)
